# baseline (speedup 1.0000x reference)
import functools

import jax
import jax.numpy as jnp
from jax import lax
from jax.experimental import pallas as pl
from jax.experimental.pallas import tpu as pltpu
from jax.experimental.pallas import tpu_sc as plsc

B = 16384
MAX_LEN = 50
TOTAL = 409600
LUT_RAW = 150
LUT_PAD = 160

NC = 2
NS = 16
NW = NC * NS
RPW = B // NW
BLK = 128
NBLK = TOTAL // BLK + 2
NROUND = 2
RPR = RPW // NROUND
NIDX = 2 * RPR
CHUNK = 128
NCHUNK = NIDX // CHUNK
CU_TILE = RPW + 8
OUT_STRIDE = 64


def _sc_body(tok_hbm, cu_hbm, lut_hbm, out_hbm,
             cu_v, lut_v, idx2d, wnd, stage, sem):
    wid = lax.axis_index("s") * NC + lax.axis_index("c")
    row0 = wid * RPW

    pltpu.sync_copy(cu_hbm.at[pl.ds(row0, CU_TILE)], cu_v)
    pltpu.sync_copy(lut_hbm, lut_v)

    iota = lax.iota(jnp.int32, 16)

    for h in range(NROUND):
        lr0 = h * RPR

        @plsc.parallel_loop(0, RPR // 16, unroll=2)
        def build(g):
            r0 = lr0 + g * 16
            s = cu_v[pl.ds(r0, 16)]
            blk = lax.shift_right_logical(s, 7)
            pos = (jnp.full((16,), g * 16, dtype=jnp.int32) + iota) * 2
            plsc.store_scatter(
                idx2d, [lax.shift_right_logical(pos, 7), pos & 127], blk)
            pos1 = pos + 1
            plsc.store_scatter(
                idx2d, [lax.shift_right_logical(pos1, 7), pos1 & 127],
                blk + 1)

        for k in range(NCHUNK):
            pltpu.make_async_copy(
                tok_hbm.at[idx2d.at[k]],
                wnd.at[pl.ds(k * CHUNK, CHUNK)], sem).start()
        for k in range(NCHUNK):
            pltpu.make_async_copy(
                tok_hbm.at[idx2d.at[0]],
                wnd.at[pl.ds(0, CHUNK)], sem).wait()

        @plsc.parallel_loop(0, RPR, unroll=8)
        def lookup(lr):
            rv = jnp.full((16,), lr0 + lr, dtype=jnp.int32)
            lv = jnp.full((16,), lr, dtype=jnp.int32)
            s = plsc.load_gather(cu_v, [rv])
            e = plsc.load_gather(cu_v, [rv + 1])
            d = s & 127
            base = (jnp.full((16,), lr0 + lr, dtype=jnp.int32)) * MAX_LEN
            for c in range(4):
                j = iota + (c * 16)
                w = d + j
                tok = plsc.load_gather(
                    wnd, [lv * 2 + lax.shift_right_logical(w, 7), w & 127])
                val = plsc.load_gather(lut_v, [tok])
                val = jnp.where(s + j < e, val, 0)
                if c < 3:
                    plsc.store_scatter(stage, [base + j], val)
                else:
                    plsc.store_scatter(stage, [base + j], val,
                                       mask=j < MAX_LEN)

    pltpu.sync_copy(stage, out_hbm.at[pl.ds(row0 * MAX_LEN, RPW * MAX_LEN)])


@functools.partial(jax.jit, static_argnames=())
def _run(tok_pad, cu_pad, lut_pad):
    mesh = plsc.VectorSubcoreMesh(core_axis_name="c", subcore_axis_name="s")
    f = pl.kernel(
        _sc_body,
        out_type=jax.ShapeDtypeStruct((B * MAX_LEN,), jnp.int32),
        mesh=mesh,
        scratch_types=[
            pltpu.VMEM((CU_TILE,), jnp.int32),
            pltpu.VMEM((LUT_PAD,), jnp.int32),
            pltpu.VMEM((NCHUNK, CHUNK), jnp.int32),
            pltpu.VMEM((NIDX, BLK), jnp.int32),
            pltpu.VMEM((RPW * MAX_LEN,), jnp.int32),
            pltpu.SemaphoreType.DMA,
        ],
        compiler_params=pltpu.CompilerParams(needs_layout_passes=False),
    )
    return f(tok_pad, cu_pad, lut_pad)


def kernel(token_ids, cu_seqlens, lut):
    tok_pad = jnp.concatenate(
        [token_ids, jnp.zeros((NBLK * BLK - TOTAL,), dtype=jnp.int32)])
    tok_pad = tok_pad.reshape(NBLK, BLK)
    cu_pad = jnp.concatenate(
        [cu_seqlens, jnp.full(((NW - 1) * RPW + CU_TILE - (B + 1),), TOTAL,
                              dtype=jnp.int32)])
    lut_pad = jnp.concatenate(
        [lut, jnp.zeros((LUT_PAD - LUT_RAW,), dtype=jnp.int32)])
    return _run(tok_pad, cu_pad, lut_pad).reshape(B, MAX_LEN)


# speedup vs baseline: 1.0834x; 1.0834x over previous
import functools

import jax
import jax.numpy as jnp
from jax import lax
from jax.experimental import pallas as pl
from jax.experimental.pallas import tpu as pltpu
from jax.experimental.pallas import tpu_sc as plsc

B = 16384
MAX_LEN = 50
TOTAL = 409600
LUT_RAW = 150
LUT_PAD = 160

NC = 2
NS = 16
NW = NC * NS
RPW = B // NW
BLK = 128
NBLK = TOTAL // BLK + 2
NROUND = 4
RPR = RPW // NROUND
NIDX = 2 * RPR
CHUNK = 128
NCHUNK = NIDX // CHUNK
CU_TILE = RPW + 8


def _sc_body(tok_hbm, cu_hbm, lut_hbm, out_hbm,
             cu_v, lut_v, idx_a, idx_b, wnd_a, wnd_b, stage,
             sem_a, sem_b, sem_out):
    wid = lax.axis_index("s") * NC + lax.axis_index("c")
    row0 = wid * RPW

    pltpu.sync_copy(cu_hbm.at[pl.ds(row0, CU_TILE)], cu_v)
    pltpu.sync_copy(lut_hbm, lut_v)

    iota = lax.iota(jnp.int32, 16)
    idx_bufs = (idx_a, idx_b)
    wnd_bufs = (wnd_a, wnd_b)
    sems = (sem_a, sem_b)

    def build(h):
        lr0 = h * RPR
        idx2d = idx_bufs[h % 2]

        @plsc.parallel_loop(0, RPR // 16, unroll=2)
        def _(g):
            s = cu_v[pl.ds(lr0 + g * 16, 16)]
            blk = lax.shift_right_logical(s, 7)
            pos = (jnp.full((16,), g * 16, dtype=jnp.int32) + iota) * 2
            plsc.store_scatter(
                idx2d, [lax.shift_right_logical(pos, 7), pos & 127], blk)
            pos1 = pos + 1
            plsc.store_scatter(
                idx2d, [lax.shift_right_logical(pos1, 7), pos1 & 127],
                blk + 1)

    def fire(h):
        idx2d, wnd, sem = idx_bufs[h % 2], wnd_bufs[h % 2], sems[h % 2]
        for k in range(NCHUNK):
            pltpu.make_async_copy(
                tok_hbm.at[idx2d.at[k]],
                wnd.at[pl.ds(k * CHUNK, CHUNK)], sem).start()

    def drain(h):
        idx2d, wnd, sem = idx_bufs[h % 2], wnd_bufs[h % 2], sems[h % 2]
        for k in range(NCHUNK):
            pltpu.make_async_copy(
                tok_hbm.at[idx2d.at[0]],
                wnd.at[pl.ds(0, CHUNK)], sem).wait()

    def lookup(h):
        lr0 = h * RPR
        wnd = wnd_bufs[h % 2]

        @plsc.parallel_loop(0, RPR, unroll=4)
        def _(lr):
            rv = jnp.full((16,), lr0 + lr, dtype=jnp.int32)
            lv = jnp.full((16,), lr, dtype=jnp.int32)
            s = plsc.load_gather(cu_v, [rv])
            e = plsc.load_gather(cu_v, [rv + 1])
            d = s & 127
            base = rv * MAX_LEN
            for c in range(4):
                j = iota + (c * 16)
                w = d + j
                tok = plsc.load_gather(
                    wnd, [lv * 2 + lax.shift_right_logical(w, 7), w & 127])
                val = plsc.load_gather(lut_v, [tok])
                val = jnp.where(s + j < e, val, 0)
                if c < 3:
                    plsc.store_scatter(stage, [base + j], val)
                else:
                    plsc.store_scatter(stage, [base + j], val,
                                       mask=j < MAX_LEN)

    def flush(h):
        off = h * RPR * MAX_LEN
        pltpu.make_async_copy(
            stage.at[pl.ds(off, RPR * MAX_LEN)],
            out_hbm.at[pl.ds(row0 * MAX_LEN + off, RPR * MAX_LEN)],
            sem_out).start()

    build(0)
    fire(0)
    for h in range(NROUND):
        if h + 1 < NROUND:
            build(h + 1)
            fire(h + 1)
        drain(h)
        lookup(h)
        flush(h)
    for h in range(NROUND):
        pltpu.make_async_copy(
            stage.at[pl.ds(0, RPR * MAX_LEN)],
            out_hbm.at[pl.ds(row0 * MAX_LEN, RPR * MAX_LEN)],
            sem_out).wait()


@functools.partial(jax.jit, static_argnames=())
def _run(tok_pad, cu_pad, lut_pad):
    mesh = plsc.VectorSubcoreMesh(core_axis_name="c", subcore_axis_name="s")
    f = pl.kernel(
        _sc_body,
        out_type=jax.ShapeDtypeStruct((B * MAX_LEN,), jnp.int32),
        mesh=mesh,
        scratch_types=[
            pltpu.VMEM((CU_TILE,), jnp.int32),
            pltpu.VMEM((LUT_PAD,), jnp.int32),
            pltpu.VMEM((NCHUNK, CHUNK), jnp.int32),
            pltpu.VMEM((NCHUNK, CHUNK), jnp.int32),
            pltpu.VMEM((NIDX, BLK), jnp.int32),
            pltpu.VMEM((NIDX, BLK), jnp.int32),
            pltpu.VMEM((RPW * MAX_LEN,), jnp.int32),
            pltpu.SemaphoreType.DMA,
            pltpu.SemaphoreType.DMA,
            pltpu.SemaphoreType.DMA,
        ],
        compiler_params=pltpu.CompilerParams(needs_layout_passes=False),
    )
    return f(tok_pad, cu_pad, lut_pad)


def kernel(token_ids, cu_seqlens, lut):
    tok_pad = jnp.concatenate(
        [token_ids, jnp.zeros((NBLK * BLK - TOTAL,), dtype=jnp.int32)])
    tok_pad = tok_pad.reshape(NBLK, BLK)
    cu_pad = jnp.concatenate(
        [cu_seqlens, jnp.full(((NW - 1) * RPW + CU_TILE - (B + 1),), TOTAL,
                              dtype=jnp.int32)])
    lut_pad = jnp.concatenate(
        [lut, jnp.zeros((LUT_PAD - LUT_RAW,), dtype=jnp.int32)])
    return _run(tok_pad, cu_pad, lut_pad).reshape(B, MAX_LEN)


# speedup vs baseline: 1.1103x; 1.0248x over previous
import functools

import jax
import jax.numpy as jnp
from jax import lax
from jax.experimental import pallas as pl
from jax.experimental.pallas import tpu as pltpu
from jax.experimental.pallas import tpu_sc as plsc

B = 16384
MAX_LEN = 50
TOTAL = 409600
LUT_RAW = 150
LUT_PAD = 160

NC = 2
NS = 16
NW = NC * NS
RPW = B // NW
BLK = 128
NBLK = TOTAL // BLK + 2
NROUND = 4
RPR = RPW // NROUND
NIDX = 2 * RPR
CHUNK = 128
NCHUNK = NIDX // CHUNK
CU_TILE = RPW + 16


def _sc_body(tok_hbm, cu_hbm, lut_hbm, out_hbm,
             cu_v, lut_v, idx_a, idx_b, wnd_a, wnd_b, stage,
             sem_a, sem_b, sem_out):
    wid = lax.axis_index("s") * NC + lax.axis_index("c")
    row0 = wid * RPW

    pltpu.sync_copy(cu_hbm.at[pl.ds(row0, CU_TILE)], cu_v)
    pltpu.sync_copy(lut_hbm, lut_v)

    iota = lax.iota(jnp.int32, 16)
    idx_bufs = (idx_a, idx_b)
    wnd_bufs = (wnd_a, wnd_b)
    sems = (sem_a, sem_b)

    def build(h):
        lr0 = h * RPR
        idx2d = idx_bufs[h % 2]

        @plsc.parallel_loop(0, RPR // 16, unroll=2)
        def _(g):
            s = cu_v[pl.ds(lr0 + g * 16, 16)]
            blk = lax.shift_right_logical(s, 7)
            idx2d[0, pl.ds(g * 16, 16)] = blk
            idx2d[1, pl.ds(g * 16, 16)] = blk + 1

    def fire(h):
        idx2d, wnd, sem = idx_bufs[h % 2], wnd_bufs[h % 2], sems[h % 2]
        for k in range(NCHUNK):
            pltpu.make_async_copy(
                tok_hbm.at[idx2d.at[k]],
                wnd.at[:, pl.ds(k * BLK, BLK)], sem).start()

    def drain(h):
        idx2d, wnd, sem = idx_bufs[h % 2], wnd_bufs[h % 2], sems[h % 2]
        for k in range(NCHUNK):
            pltpu.make_async_copy(
                tok_hbm.at[idx2d.at[0]],
                wnd.at[:, pl.ds(0, BLK)], sem).wait()

    def lookup(h):
        lr0 = h * RPR
        wnd = wnd_bufs[h % 2]

        @plsc.parallel_loop(0, RPR, unroll=4)
        def _(lr):
            r = lr0 + lr
            sv = cu_v[pl.ds(r, 16)]
            s = sv[0]
            e = sv[1]
            d = s & (BLK - 1)
            lnv = jnp.full((16,), e - s, dtype=jnp.int32)
            ob = jnp.full((16,), r * MAX_LEN, dtype=jnp.int32)
            for c in range(4):
                j = iota + (c * 16)
                tok = wnd[lr, pl.ds(d + c * 16, 16)]
                val = plsc.load_gather(lut_v, [tok])
                val = jnp.where(j < lnv, val, 0)
                if c < 3:
                    plsc.store_scatter(stage, [ob + j], val)
                else:
                    plsc.store_scatter(stage, [ob + j], val,
                                       mask=j < MAX_LEN)

    def flush(h):
        off = h * RPR * MAX_LEN
        pltpu.make_async_copy(
            stage.at[pl.ds(off, RPR * MAX_LEN)],
            out_hbm.at[pl.ds(row0 * MAX_LEN + off, RPR * MAX_LEN)],
            sem_out).start()

    build(0)
    fire(0)
    for h in range(NROUND):
        if h + 1 < NROUND:
            build(h + 1)
            fire(h + 1)
        drain(h)
        lookup(h)
        flush(h)
    for h in range(NROUND):
        pltpu.make_async_copy(
            stage.at[pl.ds(0, RPR * MAX_LEN)],
            out_hbm.at[pl.ds(row0 * MAX_LEN, RPR * MAX_LEN)],
            sem_out).wait()


@functools.partial(jax.jit, static_argnames=())
def _run(tok_pad, cu_pad, lut_pad):
    mesh = plsc.VectorSubcoreMesh(core_axis_name="c", subcore_axis_name="s")
    f = pl.kernel(
        _sc_body,
        out_type=jax.ShapeDtypeStruct((B * MAX_LEN,), jnp.int32),
        mesh=mesh,
        scratch_types=[
            pltpu.VMEM((CU_TILE,), jnp.int32),
            pltpu.VMEM((LUT_PAD,), jnp.int32),
            pltpu.VMEM((NCHUNK, CHUNK), jnp.int32),
            pltpu.VMEM((NCHUNK, CHUNK), jnp.int32),
            pltpu.VMEM((RPR, 2 * BLK), jnp.int32),
            pltpu.VMEM((RPR, 2 * BLK), jnp.int32),
            pltpu.VMEM((RPW * MAX_LEN,), jnp.int32),
            pltpu.SemaphoreType.DMA,
            pltpu.SemaphoreType.DMA,
            pltpu.SemaphoreType.DMA,
        ],
        compiler_params=pltpu.CompilerParams(needs_layout_passes=False),
    )
    return f(tok_pad, cu_pad, lut_pad)


def kernel(token_ids, cu_seqlens, lut):
    tok_pad = jnp.concatenate(
        [token_ids, jnp.zeros((NBLK * BLK - TOTAL,), dtype=jnp.int32)])
    tok_pad = tok_pad.reshape(NBLK, BLK)
    cu_pad = jnp.concatenate(
        [cu_seqlens, jnp.full(((NW - 1) * RPW + CU_TILE - (B + 1),), TOTAL,
                              dtype=jnp.int32)])
    lut_pad = jnp.concatenate(
        [lut, jnp.zeros((LUT_PAD - LUT_RAW,), dtype=jnp.int32)])
    return _run(tok_pad, cu_pad, lut_pad).reshape(B, MAX_LEN)


# speedup vs baseline: 1.1221x; 1.0106x over previous
import functools

import jax
import jax.numpy as jnp
from jax import lax
from jax.experimental import pallas as pl
from jax.experimental.pallas import tpu as pltpu
from jax.experimental.pallas import tpu_sc as plsc

B = 16384
MAX_LEN = 50
TOTAL = 409600
LUT_RAW = 150
LUT_PAD = 160

NC = 2
NS = 16
NW = NC * NS
RPW = B // NW
BLK = 128
NBLK = TOTAL // BLK
NROUND = 4
RPR = RPW // NROUND
NIDX = 2 * RPR
CHUNK = 128
NCHUNK = NIDX // CHUNK
CU_TILE = RPW + 8


def _sc_body(tok_hbm, cu_hbm, lut_hbm, out_hbm,
             cu_v, lut_v, idx_a, idx_b, wnd_a, wnd_b, stage,
             sem_a, sem_b, sem_out):
    wid = lax.axis_index("s") * NC + lax.axis_index("c")
    row0 = wid * RPW

    pltpu.sync_copy(cu_hbm.at[pl.ds(row0, CU_TILE)], cu_v)
    pltpu.sync_copy(lut_hbm, lut_v)

    iota = lax.iota(jnp.int32, 16)
    idx_bufs = (idx_a, idx_b)
    wnd_bufs = (wnd_a, wnd_b)
    sems = (sem_a, sem_b)

    def build(h):
        lr0 = h * RPR
        idx2d = idx_bufs[h % 2]

        @plsc.parallel_loop(0, RPR // 16, unroll=2)
        def _(g):
            s = cu_v[pl.ds(lr0 + g * 16, 16)]
            blk = lax.shift_right_logical(s, 7)
            blk0 = jnp.minimum(blk, NBLK - 1)
            blk1 = jnp.minimum(blk + 1, NBLK - 1)
            pos = (jnp.full((16,), g * 16, dtype=jnp.int32) + iota) * 2
            plsc.store_scatter(
                idx2d, [lax.shift_right_logical(pos, 7), pos & 127], blk0)
            pos1 = pos + 1
            plsc.store_scatter(
                idx2d, [lax.shift_right_logical(pos1, 7), pos1 & 127],
                blk1)

    def fire(h):
        idx2d, wnd, sem = idx_bufs[h % 2], wnd_bufs[h % 2], sems[h % 2]
        for k in range(NCHUNK):
            pltpu.make_async_copy(
                tok_hbm.at[idx2d.at[k]],
                wnd.at[pl.ds(k * CHUNK, CHUNK)], sem).start()

    def drain(h):
        idx2d, wnd, sem = idx_bufs[h % 2], wnd_bufs[h % 2], sems[h % 2]
        for k in range(NCHUNK):
            pltpu.make_async_copy(
                tok_hbm.at[idx2d.at[0]],
                wnd.at[pl.ds(0, CHUNK)], sem).wait()

    def lookup(h):
        lr0 = h * RPR
        wnd = wnd_bufs[h % 2]

        @plsc.parallel_loop(0, RPR, unroll=4)
        def _(lr):
            rv = jnp.full((16,), lr0 + lr, dtype=jnp.int32)
            lv = jnp.full((16,), lr, dtype=jnp.int32)
            s = plsc.load_gather(cu_v, [rv])
            e = plsc.load_gather(cu_v, [rv + 1])
            ln = e - s
            d = s & 127
            ob = rv * MAX_LEN
            for c in range(4):
                j = iota + (c * 16)
                w = d + j
                tok = plsc.load_gather(
                    wnd, [lv * 2 + lax.shift_right_logical(w, 7), w & 127])
                val = plsc.load_gather(lut_v, [tok])
                val = jnp.where(j < ln, val, 0)
                if c < 3:
                    plsc.store_scatter(stage, [ob + j], val)
                else:
                    plsc.store_scatter(stage, [ob + j], val,
                                       mask=j < MAX_LEN)

    def flush(h):
        off = h * RPR * MAX_LEN
        pltpu.make_async_copy(
            stage.at[pl.ds(off, RPR * MAX_LEN)],
            out_hbm.at[pl.ds(row0 * MAX_LEN + off, RPR * MAX_LEN)],
            sem_out).start()

    build(0)
    fire(0)
    for h in range(NROUND):
        if h + 1 < NROUND:
            build(h + 1)
            fire(h + 1)
        drain(h)
        lookup(h)
        flush(h)
    for h in range(NROUND):
        pltpu.make_async_copy(
            stage.at[pl.ds(0, RPR * MAX_LEN)],
            out_hbm.at[pl.ds(row0 * MAX_LEN, RPR * MAX_LEN)],
            sem_out).wait()


@functools.partial(jax.jit, static_argnames=())
def _run(tok_pad, cu_pad, lut_pad):
    mesh = plsc.VectorSubcoreMesh(core_axis_name="c", subcore_axis_name="s")
    f = pl.kernel(
        _sc_body,
        out_type=jax.ShapeDtypeStruct((B * MAX_LEN,), jnp.int32),
        mesh=mesh,
        scratch_types=[
            pltpu.VMEM((CU_TILE,), jnp.int32),
            pltpu.VMEM((LUT_PAD,), jnp.int32),
            pltpu.VMEM((NCHUNK, CHUNK), jnp.int32),
            pltpu.VMEM((NCHUNK, CHUNK), jnp.int32),
            pltpu.VMEM((NIDX, BLK), jnp.int32),
            pltpu.VMEM((NIDX, BLK), jnp.int32),
            pltpu.VMEM((RPW * MAX_LEN,), jnp.int32),
            pltpu.SemaphoreType.DMA,
            pltpu.SemaphoreType.DMA,
            pltpu.SemaphoreType.DMA,
        ],
        compiler_params=pltpu.CompilerParams(needs_layout_passes=False),
    )
    return f(tok_pad, cu_pad, lut_pad)


def kernel(token_ids, cu_seqlens, lut):
    tok_pad = token_ids.reshape(NBLK, BLK)
    cu_pad = jnp.concatenate(
        [cu_seqlens, jnp.full(((NW - 1) * RPW + CU_TILE - (B + 1),), TOTAL,
                              dtype=jnp.int32)])
    lut_pad = jnp.concatenate(
        [lut, jnp.zeros((LUT_PAD - LUT_RAW,), dtype=jnp.int32)])
    return _run(tok_pad, cu_pad, lut_pad).reshape(B, MAX_LEN)


# speedup vs baseline: 1.1273x; 1.0046x over previous
import functools

import jax
import jax.numpy as jnp
from jax import lax
from jax.experimental import layout as jex_layout
from jax.experimental import pallas as pl
from jax.experimental.pallas import tpu as pltpu
from jax.experimental.pallas import tpu_sc as plsc

B = 16384
MAX_LEN = 50
TOTAL = 409600
LUT_RAW = 150
LUT_PAD = 160

NC = 2
NS = 16
NW = NC * NS
RPW = B // NW
BLK = 128
NBLK = TOTAL // BLK
NROUND = 4
RPR = RPW // NROUND
NIDX = 2 * RPR
CHUNK = 128
NCHUNK = NIDX // CHUNK
CU_TILE = RPW + 8


def _sc_body(tok_hbm, cu_hbm, lut_hbm, out_hbm,
             cu_v, lut_v, idx_a, idx_b, wnd_a, wnd_b, stage,
             sem_a, sem_b, sem_out):
    wid = lax.axis_index("s") * NC + lax.axis_index("c")
    row0 = wid * RPW

    pltpu.sync_copy(cu_hbm.at[pl.ds(row0, RPW + 1)], cu_v.at[pl.ds(0, RPW + 1)])
    pltpu.sync_copy(lut_hbm, lut_v.at[pl.ds(0, LUT_RAW)])

    iota = lax.iota(jnp.int32, 16)
    idx_bufs = (idx_a, idx_b)
    wnd_bufs = (wnd_a, wnd_b)
    sems = (sem_a, sem_b)

    def build(h):
        lr0 = h * RPR
        idx2d = idx_bufs[h % 2]

        @plsc.parallel_loop(0, RPR // 16, unroll=2)
        def _(g):
            s = cu_v[pl.ds(lr0 + g * 16, 16)]
            blk = lax.shift_right_logical(s, 7)
            blk0 = jnp.minimum(blk, NBLK - 1)
            blk1 = jnp.minimum(blk + 1, NBLK - 1)
            pos = (jnp.full((16,), g * 16, dtype=jnp.int32) + iota) * 2
            plsc.store_scatter(
                idx2d, [lax.shift_right_logical(pos, 7), pos & 127], blk0)
            pos1 = pos + 1
            plsc.store_scatter(
                idx2d, [lax.shift_right_logical(pos1, 7), pos1 & 127],
                blk1)

    def fire(h):
        idx2d, wnd, sem = idx_bufs[h % 2], wnd_bufs[h % 2], sems[h % 2]
        for k in range(NCHUNK):
            pltpu.make_async_copy(
                tok_hbm.at[idx2d.at[k]],
                wnd.at[pl.ds(k * CHUNK, CHUNK)], sem).start()

    def drain(h):
        idx2d, wnd, sem = idx_bufs[h % 2], wnd_bufs[h % 2], sems[h % 2]
        for k in range(NCHUNK):
            pltpu.make_async_copy(
                tok_hbm.at[idx2d.at[0]],
                wnd.at[pl.ds(0, CHUNK)], sem).wait()

    def lookup(h):
        lr0 = h * RPR
        wnd = wnd_bufs[h % 2]

        @plsc.parallel_loop(0, RPR, unroll=4)
        def _(lr):
            rv = jnp.full((16,), lr0 + lr, dtype=jnp.int32)
            lv = jnp.full((16,), lr, dtype=jnp.int32)
            s = plsc.load_gather(cu_v, [rv])
            e = plsc.load_gather(cu_v, [rv + 1])
            ln = e - s
            d = s & 127
            ob = rv * MAX_LEN
            for c in range(4):
                j = iota + (c * 16)
                w = d + j
                tok = plsc.load_gather(
                    wnd, [lv * 2 + lax.shift_right_logical(w, 7), w & 127])
                val = plsc.load_gather(lut_v, [tok])
                val = jnp.where(j < ln, val, 0)
                if c < 3:
                    plsc.store_scatter(stage, [ob + j], val)
                else:
                    plsc.store_scatter(stage, [ob + j], val,
                                       mask=j < MAX_LEN)

    def flush(h):
        off = h * RPR * MAX_LEN
        pltpu.make_async_copy(
            stage.at[pl.ds(off, RPR * MAX_LEN)],
            out_hbm.at[pl.ds(row0 * MAX_LEN + off, RPR * MAX_LEN)],
            sem_out).start()

    build(0)
    fire(0)
    for h in range(NROUND):
        if h + 1 < NROUND:
            build(h + 1)
            fire(h + 1)
        drain(h)
        lookup(h)
        flush(h)
    for h in range(NROUND):
        pltpu.make_async_copy(
            stage.at[pl.ds(0, RPR * MAX_LEN)],
            out_hbm.at[pl.ds(row0 * MAX_LEN, RPR * MAX_LEN)],
            sem_out).wait()


def _run(tok_pad, cu_pad, lut_pad):
    mesh = plsc.VectorSubcoreMesh(core_axis_name="c", subcore_axis_name="s")
    f = pl.kernel(
        _sc_body,
        out_type=jax.ShapeDtypeStruct((B * MAX_LEN,), jnp.int32),
        mesh=mesh,
        scratch_types=[
            pltpu.VMEM((CU_TILE,), jnp.int32),
            pltpu.VMEM((LUT_PAD,), jnp.int32),
            pltpu.VMEM((NCHUNK, CHUNK), jnp.int32),
            pltpu.VMEM((NCHUNK, CHUNK), jnp.int32),
            pltpu.VMEM((NIDX, BLK), jnp.int32),
            pltpu.VMEM((NIDX, BLK), jnp.int32),
            pltpu.VMEM((RPW * MAX_LEN,), jnp.int32),
            pltpu.SemaphoreType.DMA,
            pltpu.SemaphoreType.DMA,
            pltpu.SemaphoreType.DMA,
        ],
        compiler_params=pltpu.CompilerParams(needs_layout_passes=False),
    )
    return f(tok_pad, cu_pad, lut_pad).reshape(B, MAX_LEN)


@functools.lru_cache(maxsize=1)
def _jitted_run():
    fmt = jex_layout.Format(
        jex_layout.Layout(major_to_minor=(0, 1), tiling=()),
        jax.sharding.SingleDeviceSharding(jax.devices()[0]))
    return jax.jit(_run, out_shardings=fmt)


def kernel(token_ids, cu_seqlens, lut):
    return _jitted_run()(token_ids.reshape(NBLK, BLK), cu_seqlens, lut)


# speedup vs baseline: 1.1276x; 1.0003x over previous
import functools

import jax
import jax.numpy as jnp
from jax import lax
from jax.experimental import layout as jex_layout
from jax.experimental import pallas as pl
from jax.experimental.pallas import tpu as pltpu
from jax.experimental.pallas import tpu_sc as plsc

B = 16384
MAX_LEN = 50
TOTAL = 409600
LUT_RAW = 150
LUT_PAD = 160

NC = 2
NS = 16
NW = NC * NS
RPW = B // NW
BLK = 128
NBLK = TOTAL // BLK
NROUND = 4
RPR = RPW // NROUND
NIDX = 2 * RPR
CHUNK = 128
NCHUNK = NIDX // CHUNK
CU_TILE = RPW + 8


def _sc_body(tok_hbm, cu_hbm, lut_hbm, out_hbm,
             cu_v, lut_v, idx_a, idx_b, wnd_a, wnd_b, stage,
             sem_a, sem_b, sem_out):
    wid = lax.axis_index("s") * NC + lax.axis_index("c")
    row0 = wid * RPW

    pltpu.sync_copy(cu_hbm.at[pl.ds(row0, RPW + 1)], cu_v.at[pl.ds(0, RPW + 1)])
    pltpu.sync_copy(lut_hbm, lut_v.at[pl.ds(0, LUT_RAW)])

    iota = lax.iota(jnp.int32, 16)
    idx_bufs = (idx_a, idx_b)
    wnd_bufs = (wnd_a, wnd_b)
    sems = (sem_a, sem_b)

    def build(h):
        lr0 = h * RPR
        idx2d = idx_bufs[h % 2]

        @plsc.parallel_loop(0, RPR // 16, unroll=2)
        def _(g):
            s = cu_v[pl.ds(lr0 + g * 16, 16)]
            blk = lax.shift_right_logical(s, 7)
            blk0 = jnp.minimum(blk, NBLK - 1)
            blk1 = jnp.minimum(blk + 1, NBLK - 1)
            pos = (jnp.full((16,), g * 16, dtype=jnp.int32) + iota) * 2
            plsc.store_scatter(
                idx2d, [lax.shift_right_logical(pos, 7), pos & 127], blk0)
            pos1 = pos + 1
            plsc.store_scatter(
                idx2d, [lax.shift_right_logical(pos1, 7), pos1 & 127],
                blk1)

    def fire(h):
        idx2d, wnd, sem = idx_bufs[h % 2], wnd_bufs[h % 2], sems[h % 2]
        for k in range(NCHUNK):
            pltpu.make_async_copy(
                tok_hbm.at[idx2d.at[k]],
                wnd.at[pl.ds(k * CHUNK, CHUNK)], sem).start()

    def drain(h):
        idx2d, wnd, sem = idx_bufs[h % 2], wnd_bufs[h % 2], sems[h % 2]
        for k in range(NCHUNK):
            pltpu.make_async_copy(
                tok_hbm.at[idx2d.at[0]],
                wnd.at[pl.ds(0, CHUNK)], sem).wait()

    def lookup(h):
        lr0 = h * RPR
        wnd = wnd_bufs[h % 2]

        @plsc.parallel_loop(0, RPR, unroll=4)
        def _(lr):
            rv = jnp.full((16,), lr0 + lr, dtype=jnp.int32)
            lv = jnp.full((16,), lr, dtype=jnp.int32)
            s = plsc.load_gather(cu_v, [rv])
            e = plsc.load_gather(cu_v, [rv + 1])
            ln = e - s
            d = s & 127
            ob = rv * MAX_LEN
            for c in range(4):
                j = iota + (c * 16)
                w = d + j
                tok = plsc.load_gather(
                    wnd, [lv * 2 + lax.shift_right_logical(w, 7), w & 127])
                val = plsc.load_gather(lut_v, [tok])
                val = jnp.where(j < ln, val, 0)
                if c < 3:
                    plsc.store_scatter(stage, [ob + j], val)
                else:
                    plsc.store_scatter(stage, [ob + j], val,
                                       mask=j < MAX_LEN)

    def flush(h):
        off = h * RPR * MAX_LEN
        pltpu.make_async_copy(
            stage.at[pl.ds(off, RPR * MAX_LEN)],
            out_hbm.at[pl.ds(row0 * MAX_LEN + off, RPR * MAX_LEN)],
            sem_out).start()

    build(0)
    fire(0)
    for h in range(NROUND):
        if h + 1 < NROUND:
            build(h + 1)
            fire(h + 1)
        drain(h)
        lookup(h)
        flush(h)
    for h in range(NROUND):
        pltpu.make_async_copy(
            stage.at[pl.ds(0, RPR * MAX_LEN)],
            out_hbm.at[pl.ds(row0 * MAX_LEN, RPR * MAX_LEN)],
            sem_out).wait()


def _run(tok_pad, cu_pad, lut_pad):
    mesh = plsc.VectorSubcoreMesh(core_axis_name="c", subcore_axis_name="s")
    f = pl.kernel(
        _sc_body,
        out_type=jax.ShapeDtypeStruct((B * MAX_LEN,), jnp.int32),
        mesh=mesh,
        scratch_types=[
            pltpu.VMEM((CU_TILE,), jnp.int32),
            pltpu.VMEM((LUT_PAD,), jnp.int32),
            pltpu.VMEM((NCHUNK, CHUNK), jnp.int32),
            pltpu.VMEM((NCHUNK, CHUNK), jnp.int32),
            pltpu.VMEM((NIDX, BLK), jnp.int32),
            pltpu.VMEM((NIDX, BLK), jnp.int32),
            pltpu.VMEM((RPW * MAX_LEN,), jnp.int32),
            pltpu.SemaphoreType.DMA,
            pltpu.SemaphoreType.DMA,
            pltpu.SemaphoreType.DMA,
        ],
        compiler_params=pltpu.CompilerParams(needs_layout_passes=False),
    )
    return f(tok_pad, cu_pad, lut_pad).reshape(B, MAX_LEN)


@functools.lru_cache(maxsize=1)
def _jitted_run():
    fmt = jex_layout.Format(
        jex_layout.Layout(major_to_minor=(0, 1), tiling=((1, 1),)),
        jax.sharding.SingleDeviceSharding(jax.devices()[0]))
    return jax.jit(_run, out_shardings=fmt)


def kernel(token_ids, cu_seqlens, lut):
    return _jitted_run()(token_ids.reshape(NBLK, BLK), cu_seqlens, lut)


# speedup vs baseline: 1.2145x; 1.0771x over previous
import functools

import jax
import jax.numpy as jnp
from jax import lax
from jax.experimental import layout as jex_layout
from jax.experimental import pallas as pl
from jax.experimental.pallas import tpu as pltpu
from jax.experimental.pallas import tpu_sc as plsc

B = 16384
MAX_LEN = 50
TOTAL = 409600
LUT_RAW = 150
LUT_PAD = 160

NC = 2
NS = 16
NW = NC * NS
RPW = B // NW
BLK = 128
NBLK = TOTAL // BLK
NROUND = 4
RPR = RPW // NROUND
NIDX = 2 * RPR
CHUNK = 128
NCHUNK = NIDX // CHUNK
CU_TILE = RPW + 8


def _sc_body(tok_hbm, cu_hbm, lut_hbm, out_hbm,
             cu_v, lut_v, idx_a, idx_b, wnd_a, wnd_b, stage,
             sem_a, sem_b, sem_out):
    wid = lax.axis_index("s") * NC + lax.axis_index("c")
    row0 = wid * RPW

    pltpu.sync_copy(cu_hbm.at[pl.ds(row0, RPW + 1)], cu_v.at[pl.ds(0, RPW + 1)])
    pltpu.sync_copy(lut_hbm, lut_v.at[pl.ds(0, LUT_RAW)])

    iota = lax.iota(jnp.int32, 16)
    idx_bufs = (idx_a, idx_b)
    wnd_bufs = (wnd_a, wnd_b)
    sems = (sem_a, sem_b)

    def build(h):
        lr0 = h * RPR
        idx2d = idx_bufs[h % 2]

        @plsc.parallel_loop(0, RPR // 16, unroll=2)
        def _(g):
            s = cu_v[pl.ds(lr0 + g * 16, 16)]
            blk = lax.shift_right_logical(s, 7)
            blk0 = jnp.minimum(blk, NBLK - 1)
            blk1 = jnp.minimum(blk + 1, NBLK - 1)
            pos = (jnp.full((16,), g * 16, dtype=jnp.int32) + iota) * 2
            plsc.store_scatter(
                idx2d, [lax.shift_right_logical(pos, 7), pos & 127], blk0)
            pos1 = pos + 1
            plsc.store_scatter(
                idx2d, [lax.shift_right_logical(pos1, 7), pos1 & 127],
                blk1)

    def fire(h):
        idx2d, wnd, sem = idx_bufs[h % 2], wnd_bufs[h % 2], sems[h % 2]
        for k in range(NCHUNK):
            pltpu.make_async_copy(
                tok_hbm.at[idx2d.at[k]],
                wnd.at[pl.ds(k * CHUNK, CHUNK)], sem).start()

    def drain(h):
        idx2d, wnd, sem = idx_bufs[h % 2], wnd_bufs[h % 2], sems[h % 2]
        for k in range(NCHUNK):
            pltpu.make_async_copy(
                tok_hbm.at[idx2d.at[0]],
                wnd.at[pl.ds(0, CHUNK)], sem).wait()

    def lookup(h):
        lr0 = h * RPR
        wnd = wnd_bufs[h % 2]

        @plsc.parallel_loop(0, RPR, unroll=4)
        def _(lr):
            rv = jnp.full((16,), lr0 + lr, dtype=jnp.int32)
            lv = jnp.full((16,), lr, dtype=jnp.int32)
            s = plsc.load_gather(cu_v, [rv])
            e = plsc.load_gather(cu_v, [rv + 1])
            ln = e - s
            d = s & 127
            for c in range(4):
                j = iota + (c * 16)
                w = d + j
                tok = plsc.load_gather(
                    wnd, [lv * 2 + lax.shift_right_logical(w, 7), w & 127])
                val = plsc.load_gather(lut_v, [tok])
                val = jnp.where(j < ln, val, 0)
                if c < 3:
                    plsc.store_scatter(stage, [rv, j], val)
                else:
                    plsc.store_scatter(stage, [rv, j], val,
                                       mask=j < MAX_LEN)

    def flush(h):
        lr0 = h * RPR
        pltpu.make_async_copy(
            stage.at[pl.ds(lr0, RPR), :],
            out_hbm.at[pl.ds(row0 + lr0, RPR), :],
            sem_out).start()

    build(0)
    fire(0)
    for h in range(NROUND):
        if h + 1 < NROUND:
            build(h + 1)
            fire(h + 1)
        drain(h)
        lookup(h)
        flush(h)
    for h in range(NROUND):
        pltpu.make_async_copy(
            stage.at[pl.ds(0, RPR), :],
            out_hbm.at[pl.ds(row0, RPR), :],
            sem_out).wait()


def _run(tok_pad, cu_pad, lut_pad):
    mesh = plsc.VectorSubcoreMesh(core_axis_name="c", subcore_axis_name="s")
    f = pl.kernel(
        _sc_body,
        out_type=jax.ShapeDtypeStruct((B, MAX_LEN), jnp.int32),
        mesh=mesh,
        scratch_types=[
            pltpu.VMEM((CU_TILE,), jnp.int32),
            pltpu.VMEM((LUT_PAD,), jnp.int32),
            pltpu.VMEM((NCHUNK, CHUNK), jnp.int32),
            pltpu.VMEM((NCHUNK, CHUNK), jnp.int32),
            pltpu.VMEM((NIDX, BLK), jnp.int32),
            pltpu.VMEM((NIDX, BLK), jnp.int32),
            pltpu.VMEM((RPW, MAX_LEN), jnp.int32),
            pltpu.SemaphoreType.DMA,
            pltpu.SemaphoreType.DMA,
            pltpu.SemaphoreType.DMA,
        ],
        compiler_params=pltpu.CompilerParams(
            needs_layout_passes=False, use_tc_tiling_on_sc=False),
    )
    return f(tok_pad, cu_pad, lut_pad)


@functools.lru_cache(maxsize=1)
def _jitted_run():
    fmt = jex_layout.Format(
        jex_layout.Layout(major_to_minor=(0, 1), tiling=((1, 1),)),
        jax.sharding.SingleDeviceSharding(jax.devices()[0]))
    return jax.jit(_run, out_shardings=fmt)


def kernel(token_ids, cu_seqlens, lut):
    return _jitted_run()(token_ids.reshape(NBLK, BLK), cu_seqlens, lut)
